# baseline (speedup 1.0000x reference)
import functools

import jax
import jax.numpy as jnp
from jax import lax
from jax.experimental import pallas as pl
from jax.experimental.pallas import tpu as pltpu
from jax.experimental.pallas import tpu_sc as plsc

N = 10000
HALF = 160000
D = 128
H = 64

NC, NS, L = 2, 16, 16
NW = NC * NS
E_PAD = 163840
PER_W = E_PAD // NW
CHUNK = 128
N_CHUNKS = PER_W // CHUNK
N_GROUPS = CHUNK // L


def _tc_precompute_body(ne_ref, w1_ref, b1_ref, p1_ref, p2_ref):
    ne = ne_ref[...]
    w1 = w1_ref[...]
    p1_ref[...] = jnp.dot(ne, w1[:D, :], preferred_element_type=jnp.float32) + b1_ref[...]
    p2_ref[...] = jnp.dot(ne, w1[D:, :], preferred_element_type=jnp.float32)


def _tc_precompute(node_emb, W1, b1):
    return pl.pallas_call(
        _tc_precompute_body,
        out_shape=[
            jax.ShapeDtypeStruct((N, H), jnp.float32),
            jax.ShapeDtypeStruct((N, H), jnp.float32),
        ],
    )(node_emb, W1, b1.reshape(1, H))


NBUF = 4


def _sc_edge_body(p1_hbm, p2_hbm, src_hbm, dst_hbm, ev_hbm, ns_hbm, w2_hbm,
                  nv_hbm, part_hbm,
                  src_v, dst_v, ev_v, ns_v, out_v, rows_a, rows_b,
                  acc_v, w2_v, sems):
    wid = lax.axis_index("s") * NC + lax.axis_index("c")
    base = wid * PER_W
    rbase = wid * N_CHUNKS

    pltpu.sync_copy(w2_hbm, w2_v)
    pltpu.sync_copy(src_hbm.at[pl.ds(rbase, N_CHUNKS)], src_v)
    pltpu.sync_copy(dst_hbm.at[pl.ds(rbase, N_CHUNKS)], dst_v)
    pltpu.sync_copy(ev_hbm.at[pl.ds(base, PER_W)], ev_v)
    pltpu.sync_copy(ns_hbm.at[pl.ds(base, PER_W)], ns_v)

    iota = jnp.arange(L, dtype=jnp.int32)
    zero16 = jnp.zeros((L,), jnp.float32)
    acc_v[...] = zero16
    w2q = [w2_v[pl.ds(k * L, L)] for k in range(H // L)]

    def issue(c, p):
        pltpu.async_copy(p1_hbm.at[src_v.at[c]], rows_a.at[p], sems[p])
        pltpu.async_copy(p2_hbm.at[dst_v.at[c]], rows_b.at[p], sems[p])

    def drain(p):
        pltpu.make_async_copy(p1_hbm.at[src_v.at[0]], rows_a.at[p],
                              sems[p]).wait()
        pltpu.make_async_copy(p2_hbm.at[dst_v.at[0]], rows_b.at[p],
                              sems[p]).wait()

    for p in range(NBUF):
        issue(p, p)

    def compute_chunk(c, p):
        ra = rows_a.at[p]
        rb = rows_b.at[p]

        def group_body(g, _):
            s_vec = zero16
            for ee in range(L):
                a_r = ra.at[g * L + ee]
                b_r = rb.at[g * L + ee]
                t = None
                for k in range(H // L):
                    va = a_r[pl.ds(k * L, L)]
                    vb = b_r[pl.ds(k * L, L)]
                    h = jnp.maximum(va + vb, 0.0)
                    tk = h * w2q[k]
                    t = tk if t is None else t + tk
                s = plsc.cumsum(t)[L - 1]
                s_vec = jnp.where(iota == ee, s, s_vec)
            off = c * CHUNK + g * L
            gate = s_vec + ns_v[pl.ds(off, L)]
            aug = 1.0 / (1.0 + jnp.exp(-gate))
            ids = base + off + iota
            aug_m = jnp.where(ids < HALF, aug, 0.0)
            out_v[pl.ds(off, L)] = aug * ev_v[pl.ds(off, L)]
            acc_v[...] = acc_v[...] + aug_m
            return 0

        lax.fori_loop(0, N_GROUPS, group_body, 0)

    def pair_body(c0, _):
        for p in range(NBUF):
            c = c0 * NBUF + p
            drain(p)
            compute_chunk(c, p)

            @pl.when(c + NBUF < N_CHUNKS)
            def _():
                issue(c + NBUF, p)
        return 0

    lax.fori_loop(0, N_CHUNKS // NBUF, pair_body, 0)

    pltpu.sync_copy(out_v, nv_hbm.at[pl.ds(base, PER_W)])
    pltpu.sync_copy(acc_v, part_hbm.at[wid])


_sc_edge = functools.partial(
    pl.kernel,
    out_type=[
        jax.ShapeDtypeStruct((E_PAD,), jnp.float32),
        jax.ShapeDtypeStruct((NW, L), jnp.float32),
    ],
    mesh=plsc.VectorSubcoreMesh(core_axis_name="c", subcore_axis_name="s"),
    compiler_params=pltpu.CompilerParams(needs_layout_passes=False,
                                         use_tc_tiling_on_sc=False),
    scratch_types=[
        pltpu.VMEM((NW * N_CHUNKS // NW, CHUNK), jnp.int32),
        pltpu.VMEM((NW * N_CHUNKS // NW, CHUNK), jnp.int32),
        pltpu.VMEM((PER_W,), jnp.float32),
        pltpu.VMEM((PER_W,), jnp.float32),
        pltpu.VMEM((PER_W,), jnp.float32),
        pltpu.VMEM((NBUF, CHUNK, H), jnp.float32),
        pltpu.VMEM((NBUF, CHUNK, H), jnp.float32),
        pltpu.VMEM((L,), jnp.float32),
        pltpu.VMEM((H,), jnp.float32),
        [pltpu.SemaphoreType.DMA] * NBUF,
    ],
)(_sc_edge_body)


def kernel(node_emb, edge_index, edge_vals, W1, b1, W2, b2):
    half = edge_index.shape[1] // 2
    src = edge_index[0, :half]
    dst = edge_index[1, :half]

    p1, p2 = _tc_precompute(node_emb, W1, b1)

    bias = 0.0 + 0.0001
    u = jax.random.uniform(jax.random.key(42), (half, 1), dtype=jnp.float32)
    eps = (bias - (1.0 - bias)) * u + (1.0 - bias)
    noise = (jnp.log(eps) - jnp.log(1.0 - eps)).reshape(half)
    noise = noise + b2[0]

    pad = E_PAD - half
    src_p = jnp.pad(src, (0, pad)).reshape(E_PAD // CHUNK, CHUNK)
    dst_p = jnp.pad(dst, (0, pad)).reshape(E_PAD // CHUNK, CHUNK)
    ev_p = jnp.pad(edge_vals[:half], (0, pad))
    ns_p = jnp.pad(noise, (0, pad))

    nv_p, partials = _sc_edge(p1, p2, src_p, dst_p, ev_p, ns_p,
                              W2.reshape(H))

    nv = nv_p[:half]
    mean_edge_weight = jnp.sum(partials) / half
    sym_inds = jnp.concatenate(
        [jnp.stack([src, dst]), jnp.stack([dst, src])], axis=1)
    sym_vals = jnp.concatenate([nv, nv])
    return (sym_inds, sym_vals, mean_edge_weight)


# speedup vs baseline: 1.5366x; 1.5366x over previous
import functools

import jax
import jax.numpy as jnp
from jax import lax
from jax.experimental import pallas as pl
from jax.experimental.pallas import tpu as pltpu
from jax.experimental.pallas import tpu_sc as plsc

N = 10000
HALF = 160000
D = 128
H = 64

NC, NS, L = 2, 16, 16
NW = NC * NS
E_PAD = 163840
PER_W = E_PAD // NW
CHUNK = 128
N_CHUNKS = PER_W // CHUNK
N_GROUPS = CHUNK // L


def _tc_precompute_body(ne_ref, w1_ref, b1_ref, p1_ref, p2_ref):
    ne = ne_ref[...]
    w1 = w1_ref[...]
    p1 = jnp.dot(ne, w1[:D, :], preferred_element_type=jnp.float32) + b1_ref[...]
    p2 = jnp.dot(ne, w1[D:, :], preferred_element_type=jnp.float32)
    p1_ref[...] = p1.astype(jnp.bfloat16)
    p2_ref[...] = p2.astype(jnp.bfloat16)


def _tc_precompute(node_emb, W1, b1):
    return pl.pallas_call(
        _tc_precompute_body,
        out_shape=[
            jax.ShapeDtypeStruct((N, H), jnp.bfloat16),
            jax.ShapeDtypeStruct((N, H), jnp.bfloat16),
        ],
    )(node_emb, W1, b1.reshape(1, H))


NBUF = 4


def _sc_edge_body(p1_hbm, p2_hbm, src_hbm, dst_hbm, ev_hbm, ns_hbm, w2_hbm,
                  nv_hbm, part_hbm,
                  src_v, dst_v, ev_v, ns_v, out_v, rows_a, rows_b,
                  acc_v, w2_v, sems):
    wid = lax.axis_index("s") * NC + lax.axis_index("c")
    base = wid * PER_W
    rbase = wid * N_CHUNKS

    pltpu.sync_copy(w2_hbm, w2_v)
    pltpu.sync_copy(src_hbm.at[pl.ds(rbase, N_CHUNKS)], src_v)
    pltpu.sync_copy(dst_hbm.at[pl.ds(rbase, N_CHUNKS)], dst_v)
    pltpu.sync_copy(ev_hbm.at[pl.ds(base, PER_W)], ev_v)
    pltpu.sync_copy(ns_hbm.at[pl.ds(base, PER_W)], ns_v)

    iota = jnp.arange(L, dtype=jnp.int32)
    zero16 = jnp.zeros((L,), jnp.float32)
    zero32b = jnp.zeros((2 * L,), jnp.bfloat16)
    acc_v[...] = zero16
    w2q = [w2_v[pl.ds(k * L, L)] for k in range(H // L)]

    def issue(c, p):
        pltpu.async_copy(p1_hbm.at[src_v.at[c]], rows_a.at[p], sems[p])
        pltpu.async_copy(p2_hbm.at[dst_v.at[c]], rows_b.at[p], sems[p])

    def drain(p):
        pltpu.make_async_copy(p1_hbm.at[src_v.at[0]], rows_a.at[p],
                              sems[p]).wait()
        pltpu.make_async_copy(p2_hbm.at[dst_v.at[0]], rows_b.at[p],
                              sems[p]).wait()

    for p in range(NBUF):
        issue(p, p)

    def compute_chunk(c, p):
        ra = rows_a.at[p]
        rb = rows_b.at[p]

        def group_body(g, _):
            s_vec = zero16
            for ee in range(L):
                a_r = ra.at[g * L + ee]
                b_r = rb.at[g * L + ee]
                t = None
                for k in range(H // (2 * L)):
                    va = a_r[pl.ds(k * 2 * L, 2 * L)]
                    vb = b_r[pl.ds(k * 2 * L, 2 * L)]
                    h = jnp.maximum(va + vb, zero32b)
                    u0, u1 = plsc.unpack(h, format=plsc.PackFormat.INTERLEAVED)
                    tk = u0 * w2q[2 * k] + u1 * w2q[2 * k + 1]
                    t = tk if t is None else t + tk
                s = plsc.cumsum(t)[L - 1]
                s_vec = jnp.where(iota == ee, s, s_vec)
            off = c * CHUNK + g * L
            gate = s_vec + ns_v[pl.ds(off, L)]
            aug = 1.0 / (1.0 + jnp.exp(-gate))
            ids = base + off + iota
            aug_m = jnp.where(ids < HALF, aug, 0.0)
            out_v[pl.ds(off, L)] = aug * ev_v[pl.ds(off, L)]
            acc_v[...] = acc_v[...] + aug_m
            return 0

        lax.fori_loop(0, N_GROUPS, group_body, 0)

    def pair_body(c0, _):
        for p in range(NBUF):
            c = c0 * NBUF + p
            drain(p)
            compute_chunk(c, p)

            @pl.when(c + NBUF < N_CHUNKS)
            def _():
                issue(c + NBUF, p)
        return 0

    lax.fori_loop(0, N_CHUNKS // NBUF, pair_body, 0)

    pltpu.sync_copy(out_v, nv_hbm.at[pl.ds(base, PER_W)])
    pltpu.sync_copy(acc_v, part_hbm.at[wid])


_sc_edge = functools.partial(
    pl.kernel,
    out_type=[
        jax.ShapeDtypeStruct((E_PAD,), jnp.float32),
        jax.ShapeDtypeStruct((NW, L), jnp.float32),
    ],
    mesh=plsc.VectorSubcoreMesh(core_axis_name="c", subcore_axis_name="s"),
    compiler_params=pltpu.CompilerParams(needs_layout_passes=False,
                                         use_tc_tiling_on_sc=False),
    scratch_types=[
        pltpu.VMEM((NW * N_CHUNKS // NW, CHUNK), jnp.int32),
        pltpu.VMEM((NW * N_CHUNKS // NW, CHUNK), jnp.int32),
        pltpu.VMEM((PER_W,), jnp.float32),
        pltpu.VMEM((PER_W,), jnp.float32),
        pltpu.VMEM((PER_W,), jnp.float32),
        pltpu.VMEM((NBUF, CHUNK, H), jnp.bfloat16),
        pltpu.VMEM((NBUF, CHUNK, H), jnp.bfloat16),
        pltpu.VMEM((L,), jnp.float32),
        pltpu.VMEM((H,), jnp.float32),
        [pltpu.SemaphoreType.DMA] * NBUF,
    ],
)(_sc_edge_body)


def kernel(node_emb, edge_index, edge_vals, W1, b1, W2, b2):
    half = edge_index.shape[1] // 2
    src = edge_index[0, :half]
    dst = edge_index[1, :half]

    p1, p2 = _tc_precompute(node_emb, W1, b1)

    bias = 0.0 + 0.0001
    u = jax.random.uniform(jax.random.key(42), (half, 1), dtype=jnp.float32)
    eps = (bias - (1.0 - bias)) * u + (1.0 - bias)
    noise = (jnp.log(eps) - jnp.log(1.0 - eps)).reshape(half)
    noise = noise + b2[0]

    pad = E_PAD - half
    src_p = jnp.pad(src, (0, pad)).reshape(E_PAD // CHUNK, CHUNK)
    dst_p = jnp.pad(dst, (0, pad)).reshape(E_PAD // CHUNK, CHUNK)
    ev_p = jnp.pad(edge_vals[:half], (0, pad))
    ns_p = jnp.pad(noise, (0, pad))

    w2f = W2.reshape(H)
    w2_perm = jnp.concatenate(
        [w2f[0:32][0::2], w2f[0:32][1::2], w2f[32:64][0::2], w2f[32:64][1::2]])

    nv_p, partials = _sc_edge(p1, p2, src_p, dst_p, ev_p, ns_p, w2_perm)

    nv = nv_p[:half]
    mean_edge_weight = jnp.sum(partials) / half
    sym_inds = jnp.concatenate(
        [jnp.stack([src, dst]), jnp.stack([dst, src])], axis=1)
    sym_vals = jnp.concatenate([nv, nv])
    return (sym_inds, sym_vals, mean_edge_weight)
